# baseline (speedup 1.0000x reference)
import functools

import jax
import jax.numpy as jnp
from jax import lax
from jax.experimental import pallas as pl
from jax.experimental.pallas import tpu as pltpu
from jax.experimental.pallas import tpu_sc as plsc

_BATCH = 1024
_SEQ = 200
_HID = 128
_ROWS = _BATCH * _SEQ
_NW = 32
_CHUNK = _SEQ // 2
_NCHUNK = _ROWS // _CHUNK
_CHUNKS_PER_W = _NCHUNK // _NW
_PAIRS = _CHUNKS_PER_W // 2
_LANES = 16

_mesh = plsc.VectorSubcoreMesh(core_axis_name="c", subcore_axis_name="s")


@functools.partial(
    pl.kernel,
    mesh=_mesh,
    out_type=jax.ShapeDtypeStruct((_NCHUNK, _CHUNK, _HID), jnp.float32),
    scratch_types=[
        pltpu.VMEM((_CHUNKS_PER_W, _CHUNK), jnp.int32),
        pltpu.VMEM((_SEQ, _HID), jnp.float32),
        pltpu.VMEM((_CHUNK, _HID), jnp.float32),
        pltpu.VMEM((_CHUNK, _HID), jnp.float32),
        pltpu.VMEM((_CHUNK, _HID), jnp.float32),
        pltpu.VMEM((_CHUNK, _HID), jnp.float32),
        pltpu.SemaphoreType.DMA,
        pltpu.SemaphoreType.DMA,
        pltpu.SemaphoreType.DMA,
        pltpu.SemaphoreType.DMA,
    ],
)
def _emb_lookup(x_hbm, emb_hbm, pos_hbm, out_hbm,
                idx_v, pos_v, in0, in1, out0, out1, g0, g1, w0, w1):
    wid = lax.axis_index("s") * 2 + lax.axis_index("c")
    cbase = wid * _CHUNKS_PER_W
    pltpu.sync_copy(x_hbm.at[pl.ds(cbase, _CHUNKS_PER_W)], idx_v)
    pltpu.sync_copy(pos_hbm, pos_v)

    def gather(c, buf, sem):
        return pltpu.async_copy(emb_hbm.at[idx_v.at[c]], buf, sem)

    def writeback(c, buf, sem):
        return pltpu.async_copy(buf, out_hbm.at[cbase + c], sem)

    def gather_wait(buf, sem):
        pltpu.make_async_copy(emb_hbm.at[idx_v.at[0]], buf, sem).wait()

    def writeback_wait(buf, sem):
        pltpu.make_async_copy(buf, out_hbm.at[cbase], sem).wait()

    def add_pos(src, dst, pos_off):
        def row(l, c):
            for j in range(_HID // _LANES):
                sl = pl.ds(j * _LANES, _LANES)
                dst[l, sl] = src[l, sl] + pos_v[pos_off + l, sl]
            return c
        lax.fori_loop(0, _CHUNK, row, 0)

    gather(0, in0, g0)
    gather(1, in1, g1)

    def pair_body(t, carry):
        c0 = 2 * t

        @pl.when(t > 0)
        def _():
            writeback_wait(out0, w0)

        gather_wait(in0, g0)
        add_pos(in0, out0, 0)
        writeback(c0, out0, w0)

        @pl.when(t < _PAIRS - 1)
        def _():
            gather(c0 + 2, in0, g0)

        @pl.when(t > 0)
        def _():
            writeback_wait(out1, w1)

        gather_wait(in1, g1)
        add_pos(in1, out1, _CHUNK)
        writeback(c0 + 1, out1, w1)

        @pl.when(t < _PAIRS - 1)
        def _():
            gather(c0 + 3, in1, g1)

        return carry

    lax.fori_loop(0, _PAIRS, pair_body, 0)

    writeback_wait(out0, w0)
    writeback_wait(out1, w1)


def kernel(x, emb_table, pos_table):
    x2 = x.reshape(_NCHUNK, _CHUNK).astype(jnp.int32)
    out = _emb_lookup(x2, emb_table, pos_table)
    return out.reshape(_BATCH, _SEQ, _HID)


# speedup vs baseline: 1.6944x; 1.6944x over previous
import functools

import jax
import jax.numpy as jnp
from jax import lax
from jax.experimental import pallas as pl
from jax.experimental.pallas import tpu as pltpu
from jax.experimental.pallas import tpu_sc as plsc

_BATCH = 1024
_SEQ = 200
_HID = 128
_NW = 32
_BATCH_PER_W = _BATCH // _NW
_CHUNK = 100
_NCHUNK = _BATCH * _SEQ // _CHUNK
_CHUNKS_PER_W = _NCHUNK // _NW
_LANES = 16

_mesh = plsc.VectorSubcoreMesh(core_axis_name="c", subcore_axis_name="s")


@functools.partial(
    pl.kernel,
    mesh=_mesh,
    out_type=jax.ShapeDtypeStruct((_BATCH, _SEQ, _HID), jnp.float32),
    scratch_types=[
        pltpu.VMEM((_CHUNKS_PER_W, _CHUNK), jnp.int32),
        pltpu.VMEM((_SEQ, _HID), jnp.float32),
        pltpu.VMEM((_SEQ, _HID), jnp.float32),
        pltpu.VMEM((_SEQ, _HID), jnp.float32),
        pltpu.SemaphoreType.DMA,
        pltpu.SemaphoreType.DMA,
        pltpu.SemaphoreType.DMA,
        pltpu.SemaphoreType.DMA,
    ],
)
def _emb_lookup(x_hbm, emb_hbm, pos_hbm, out_hbm,
                idx_v, pos_v, buf0, buf1, g0, g1, w0, w1):
    wid = lax.axis_index("s") * 2 + lax.axis_index("c")
    bbase = wid * _BATCH_PER_W
    cbase = wid * _CHUNKS_PER_W
    pltpu.sync_copy(x_hbm.at[pl.ds(cbase, _CHUNKS_PER_W)], idx_v)
    pltpu.sync_copy(pos_hbm, pos_v)

    def gather(b, buf, sem):
        pltpu.async_copy(
            emb_hbm.at[idx_v.at[2 * b]], buf.at[pl.ds(0, _CHUNK)], sem)
        pltpu.async_copy(
            emb_hbm.at[idx_v.at[2 * b + 1]], buf.at[pl.ds(_CHUNK, _CHUNK)],
            sem)

    def writeback(b, buf, sem):
        return pltpu.async_copy(buf, out_hbm.at[bbase + b], sem)

    def gather_wait(buf, sem):
        pltpu.make_async_copy(emb_hbm.at[pl.ds(0, _SEQ)], buf, sem).wait()

    def writeback_wait(buf, sem):
        pltpu.make_async_copy(buf, out_hbm.at[bbase], sem).wait()

    def add_pos(buf):
        def row(l, c):
            for j in range(_HID // _LANES):
                sl = pl.ds(j * _LANES, _LANES)
                buf[l, sl] = buf[l, sl] + pos_v[l, sl]
            return c
        lax.fori_loop(0, _SEQ, row, 0)

    gather(0, buf0, g0)
    gather(1, buf1, g1)

    def pair_body(t, carry):
        b0 = 2 * t

        gather_wait(buf0, g0)
        add_pos(buf0)
        writeback(b0, buf0, w0)

        gather_wait(buf1, g1)
        add_pos(buf1)
        writeback(b0 + 1, buf1, w1)

        @pl.when(t < _BATCH_PER_W // 2 - 1)
        def _():
            writeback_wait(buf0, w0)
            gather(b0 + 2, buf0, g0)
            writeback_wait(buf1, w1)
            gather(b0 + 3, buf1, g1)

        return carry

    lax.fori_loop(0, _BATCH_PER_W // 2, pair_body, 0)

    writeback_wait(buf0, w0)
    writeback_wait(buf1, w1)


def kernel(x, emb_table, pos_table):
    x2 = x.astype(jnp.int32).reshape(_NCHUNK, _CHUNK)
    return _emb_lookup(x2, emb_table, pos_table)


# speedup vs baseline: 1.8800x; 1.1095x over previous
import functools

import jax
import jax.numpy as jnp
from jax import lax
from jax.experimental import pallas as pl
from jax.experimental.pallas import tpu as pltpu
from jax.experimental.pallas import tpu_sc as plsc

_BATCH = 1024
_SEQ = 200
_HID = 128
_NW = 32
_BATCH_PER_W = _BATCH // _NW
_CHUNK = 100
_NCHUNK = _BATCH * _SEQ // _CHUNK
_CHUNKS_PER_W = _NCHUNK // _NW
_LANES = 16

_mesh = plsc.VectorSubcoreMesh(core_axis_name="c", subcore_axis_name="s")


@functools.partial(
    pl.kernel,
    mesh=_mesh,
    out_type=jax.ShapeDtypeStruct((_BATCH, _SEQ, _HID), jnp.float32),
    scratch_types=[
        pltpu.VMEM((_CHUNKS_PER_W, _CHUNK), jnp.int32),
        pltpu.VMEM((_SEQ, _HID), jnp.float32),
        pltpu.VMEM((_CHUNK, _HID), jnp.float32),
        pltpu.VMEM((_CHUNK, _HID), jnp.float32),
        pltpu.VMEM((_SEQ, _HID), jnp.float32),
        pltpu.VMEM((_SEQ, _HID), jnp.float32),
        pltpu.SemaphoreType.DMA,
        pltpu.SemaphoreType.DMA,
        pltpu.SemaphoreType.DMA,
        pltpu.SemaphoreType.DMA,
    ],
)
def _emb_lookup(x_hbm, emb_hbm, pos_hbm, out_hbm,
                idx_v, pos_v, in0, in1, out0, out1, g0, g1, w0, w1):
    wid = lax.axis_index("s") * 2 + lax.axis_index("c")
    bbase = wid * _BATCH_PER_W
    cbase = wid * _CHUNKS_PER_W
    pltpu.sync_copy(x_hbm.at[pl.ds(cbase, _CHUNKS_PER_W)], idx_v)
    pltpu.sync_copy(pos_hbm, pos_v)

    def gather(c, buf, sem):
        pltpu.async_copy(emb_hbm.at[idx_v.at[c]], buf, sem)

    def gather_wait(buf, sem):
        pltpu.make_async_copy(emb_hbm.at[idx_v.at[0]], buf, sem).wait()

    def writeback(b, buf, sem):
        pltpu.async_copy(buf, out_hbm.at[bbase + b], sem)

    def writeback_wait(buf, sem):
        pltpu.make_async_copy(buf, out_hbm.at[bbase], sem).wait()

    def add_pos(src, dst, half):
        off = half * _CHUNK

        def row(l, c):
            for j in range(_HID // _LANES):
                sl = pl.ds(j * _LANES, _LANES)
                dst[off + l, sl] = src[l, sl] + pos_v[off + l, sl]
            return c
        lax.fori_loop(0, _CHUNK, row, 0)

    gather(0, in0, g0)
    gather(1, in1, g1)

    def seq_sub(t, b, obuf, wsem, guard_next):
        gather_wait(in0, g0)

        @pl.when(t > 0)
        def _():
            writeback_wait(obuf, wsem)

        add_pos(in0, obuf, 0)

        @pl.when(guard_next)
        def _():
            gather(2 * b + 2, in0, g0)

        gather_wait(in1, g1)
        add_pos(in1, obuf, 1)

        @pl.when(guard_next)
        def _():
            gather(2 * b + 3, in1, g1)

        writeback(b, obuf, wsem)

    def pair_body(t, carry):
        b0 = 2 * t
        seq_sub(t, b0, out0, w0, True)
        seq_sub(t, b0 + 1, out1, w1, t < _BATCH_PER_W // 2 - 1)
        return carry

    lax.fori_loop(0, _BATCH_PER_W // 2, pair_body, 0)

    writeback_wait(out0, w0)
    writeback_wait(out1, w1)


def kernel(x, emb_table, pos_table):
    x2 = x.astype(jnp.int32).reshape(_NCHUNK, _CHUNK)
    return _emb_lookup(x2, emb_table, pos_table)


# speedup vs baseline: 1.8876x; 1.0040x over previous
import functools

import jax
import jax.numpy as jnp
from jax import lax
from jax.experimental import pallas as pl
from jax.experimental.pallas import tpu as pltpu
from jax.experimental.pallas import tpu_sc as plsc

_BATCH = 1024
_SEQ = 200
_HID = 128
_NW = 32
_BATCH_PER_W = _BATCH // _NW
_CHUNK = 100
_NCHUNK = _BATCH * _SEQ // _CHUNK
_CHUNKS_PER_W = _NCHUNK // _NW
_LANES = 16

_mesh = plsc.VectorSubcoreMesh(core_axis_name="c", subcore_axis_name="s")


@functools.partial(
    pl.kernel,
    mesh=_mesh,
    out_type=jax.ShapeDtypeStruct((_BATCH, _SEQ, _HID), jnp.float32),
    scratch_types=[
        pltpu.VMEM((_CHUNKS_PER_W, _CHUNK), jnp.int32),
        pltpu.VMEM((_SEQ, _HID), jnp.float32),
        pltpu.VMEM((_CHUNK, _HID), jnp.float32),
        pltpu.VMEM((_CHUNK, _HID), jnp.float32),
        pltpu.VMEM((_SEQ, _HID), jnp.float32),
        pltpu.VMEM((_SEQ, _HID), jnp.float32),
        pltpu.SemaphoreType.DMA,
        pltpu.SemaphoreType.DMA,
        pltpu.SemaphoreType.DMA,
        pltpu.SemaphoreType.DMA,
    ],
)
def _emb_lookup(x_hbm, emb_hbm, pos_hbm, out_hbm,
                idx_v, pos_v, in0, in1, out0, out1, g0, g1, w0, w1):
    wid = lax.axis_index("s") * 2 + lax.axis_index("c")
    bbase = wid * _BATCH_PER_W
    cbase = wid * _CHUNKS_PER_W
    pltpu.sync_copy(x_hbm.at[pl.ds(cbase, 8)], idx_v.at[pl.ds(0, 8)])

    def gather(c, buf, sem):
        pltpu.async_copy(emb_hbm.at[idx_v.at[c]], buf, sem)

    def gather_wait(buf, sem):
        pltpu.make_async_copy(emb_hbm.at[idx_v.at[0]], buf, sem).wait()

    def writeback(b, buf, sem):
        pltpu.async_copy(buf, out_hbm.at[bbase + b], sem)

    def writeback_wait(buf, sem):
        pltpu.make_async_copy(buf, out_hbm.at[bbase], sem).wait()

    def add_pos(src, dst, half):
        off = half * _CHUNK

        def row(l, c):
            for j in range(_HID // _LANES):
                sl = pl.ds(j * _LANES, _LANES)
                dst[off + l, sl] = src[l, sl] + pos_v[off + l, sl]
            return c
        lax.fori_loop(0, _CHUNK, row, 0)

    gather(0, in0, g0)
    gather(1, in1, g1)
    pltpu.sync_copy(
        x_hbm.at[pl.ds(cbase + 8, _CHUNKS_PER_W - 8)],
        idx_v.at[pl.ds(8, _CHUNKS_PER_W - 8)])
    pltpu.sync_copy(pos_hbm, pos_v)

    def seq_sub(t, b, obuf, wsem, guard_next):
        gather_wait(in0, g0)

        @pl.when(t > 0)
        def _():
            writeback_wait(obuf, wsem)

        add_pos(in0, obuf, 0)

        @pl.when(guard_next)
        def _():
            gather(2 * b + 2, in0, g0)

        gather_wait(in1, g1)
        add_pos(in1, obuf, 1)

        @pl.when(guard_next)
        def _():
            gather(2 * b + 3, in1, g1)

        writeback(b, obuf, wsem)

    def pair_body(t, carry):
        b0 = 2 * t
        seq_sub(t, b0, out0, w0, True)
        seq_sub(t, b0 + 1, out1, w1, t < _BATCH_PER_W // 2 - 1)
        return carry

    lax.fori_loop(0, _BATCH_PER_W // 2, pair_body, 0)

    writeback_wait(out0, w0)
    writeback_wait(out1, w1)


def kernel(x, emb_table, pos_table):
    x2 = x.astype(jnp.int32).reshape(_NCHUNK, _CHUNK)
    return _emb_lookup(x2, emb_table, pos_table)
